# baseline (speedup 1.0000x reference)
import functools

import jax
import jax.numpy as jnp
from jax import lax
from jax.experimental import pallas as pl
from jax.experimental.pallas import tpu as pltpu
from jax.experimental.pallas import tpu_sc as plsc

_NC = 2
_NS = 16
_NW = _NC * _NS
_D = 512
_N = 512
_G = 32 * 3
_ROWS = _G * _N
_BPW = _ROWS // _NW
_C = 64
_NCH = _BPW // _C
_NBUF = 3
_NPP = _NCH // _NBUF


def _gather_rows():
    mesh = plsc.VectorSubcoreMesh(core_axis_name="c", subcore_axis_name="s")

    @functools.partial(
        pl.kernel,
        mesh=mesh,
        out_type=jax.ShapeDtypeStruct((_ROWS, _D), jnp.float32),
        scratch_types=[
            pltpu.VMEM((_NCH, _C), jnp.int32),
        ]
        + [pltpu.VMEM((_C, _D), jnp.float32) for _ in range(_NBUF)]
        + [pltpu.SemaphoreType.DMA for _ in range(2 * _NBUF)],
    )
    def k(tbl_hbm, idx_hbm, out_hbm, idx_v, *bufs_and_sems):
        rows = bufs_and_sems[:_NBUF]
        gsem = bufs_and_sems[_NBUF:2 * _NBUF]
        ssem = bufs_and_sems[2 * _NBUF:]
        wid = lax.axis_index("s") * _NC + lax.axis_index("c")
        base = wid * _BPW
        pltpu.sync_copy(idx_hbm.at[wid], idx_v)

        for b in range(_NBUF):
            pltpu.async_copy(tbl_hbm.at[idx_v.at[b]], rows[b], gsem[b])

        def body(p, carry):
            for b in range(_NBUF):
                j = _NBUF * p + b
                pltpu.make_async_copy(
                    tbl_hbm.at[idx_v.at[j]], rows[b], gsem[b]).wait()
                pltpu.async_copy(
                    rows[b], out_hbm.at[pl.ds(base + j * _C, _C)], ssem[b])

            @pl.when(p < _NPP - 1)
            def _():
                for b in range(_NBUF):
                    j = _NBUF * p + b
                    pltpu.make_async_copy(
                        rows[b], out_hbm.at[pl.ds(base, _C)], ssem[b]).wait()
                    pltpu.async_copy(
                        tbl_hbm.at[idx_v.at[j + _NBUF]], rows[b], gsem[b])

            return carry

        lax.fori_loop(0, _NPP, body, 0)

        for b in range(_NBUF):
            pltpu.make_async_copy(
                rows[b], out_hbm.at[pl.ds(base, _C)], ssem[b]).wait()

    return k


_KERNEL = _gather_rows()


@jax.jit
def kernel(img):
    perm = jax.random.permutation(jax.random.key(42), _N).astype(jnp.int32)
    gidx = (jnp.arange(_G, dtype=jnp.int32)[:, None] * _N + perm[None, :])
    gidx = gidx.reshape(_NW, _NCH, _C)
    tbl = img.reshape(_ROWS, _D)
    out = _KERNEL(tbl, gidx)
    return out.reshape(img.shape)


# speedup vs baseline: 1.0811x; 1.0811x over previous
import functools

import jax
import jax.numpy as jnp
from jax import lax
from jax.experimental import pallas as pl
from jax.experimental.pallas import tpu as pltpu
from jax.experimental.pallas import tpu_sc as plsc

_NC = 2
_NS = 16
_NW = _NC * _NS
_D = 512
_N = 512
_G = 32 * 3
_ROWS = _G * _N
_BPW = _ROWS // _NW
_C = 96
_NCH = _BPW // _C
_NP = _NCH // 2


def _gather_rows():
    mesh = plsc.VectorSubcoreMesh(core_axis_name="c", subcore_axis_name="s")

    @functools.partial(
        pl.kernel,
        mesh=mesh,
        out_type=jax.ShapeDtypeStruct((_ROWS, _D), jnp.float32),
        scratch_types=[
            pltpu.VMEM((_NCH, _C), jnp.int32),
            pltpu.VMEM((_C, _D), jnp.float32),
            pltpu.VMEM((_C, _D), jnp.float32),
            pltpu.SemaphoreType.DMA,
            pltpu.SemaphoreType.DMA,
            pltpu.SemaphoreType.DMA,
            pltpu.SemaphoreType.DMA,
        ],
    )
    def k(tbl_hbm, idx_hbm, out_hbm, idx_v, rows0, rows1, gs0, gs1, ss0, ss1):
        wid = lax.axis_index("s") * _NC + lax.axis_index("c")
        base = wid * _BPW
        pltpu.sync_copy(idx_hbm.at[wid], idx_v)

        pltpu.async_copy(tbl_hbm.at[idx_v.at[0]], rows0, gs0)

        def body(p, carry):
            j0 = 2 * p
            j1 = j0 + 1

            @pl.when(p > 0)
            def _():
                pltpu.make_async_copy(
                    rows1, out_hbm.at[pl.ds(base, _C)], ss1).wait()

            pltpu.async_copy(tbl_hbm.at[idx_v.at[j1]], rows1, gs1)

            pltpu.make_async_copy(
                tbl_hbm.at[idx_v.at[j0]], rows0, gs0).wait()
            pltpu.async_copy(rows0, out_hbm.at[pl.ds(base + j0 * _C, _C)], ss0)

            @pl.when(p < _NP - 1)
            def _():
                pltpu.make_async_copy(
                    rows0, out_hbm.at[pl.ds(base, _C)], ss0).wait()
                pltpu.async_copy(
                    tbl_hbm.at[idx_v.at[j0 + 2]], rows0, gs0)

            pltpu.make_async_copy(
                tbl_hbm.at[idx_v.at[j1]], rows1, gs1).wait()
            pltpu.async_copy(rows1, out_hbm.at[pl.ds(base + j1 * _C, _C)], ss1)
            return carry

        lax.fori_loop(0, _NP, body, 0)

        pltpu.make_async_copy(rows0, out_hbm.at[pl.ds(base, _C)], ss0).wait()
        pltpu.make_async_copy(rows1, out_hbm.at[pl.ds(base, _C)], ss1).wait()

    return k


_KERNEL = _gather_rows()


@jax.jit
def kernel(img):
    perm = jax.random.permutation(jax.random.key(42), _N).astype(jnp.int32)
    gidx = (jnp.arange(_G, dtype=jnp.int32)[:, None] * _N + perm[None, :])
    gidx = gidx.reshape(_NW, _NCH, _C)
    tbl = img.reshape(_ROWS, _D)
    out = _KERNEL(tbl, gidx)
    return out.reshape(img.shape)
